# baseline (speedup 1.0000x reference)
import functools

import jax
import jax.numpy as jnp
from jax import lax
from jax.experimental import pallas as pl
from jax.experimental.pallas import tpu as pltpu
from jax.experimental.pallas import tpu_sc as plsc

N_TOK = 8192
N_EMBED = 8192
EMBED_DIM = 64
BETA = 0.25
TN = 1024
GRID = N_TOK // TN


_KC = 256
_NKC = N_EMBED // _KC


def _argmax_body(z_ref, w_ref, idx_ref, wpad_ref, ncb_ref):
    step = pl.program_id(0)

    @pl.when(step == 0)
    def _():
        w = w_ref[...]
        ncb_ref[...] = w / jnp.maximum(
            jnp.sqrt(jnp.sum(w * w, axis=1, keepdims=True)), 1e-12)
        wpad_ref[:, :EMBED_DIM] = w
        wpad_ref[:, EMBED_DIM:] = jnp.zeros_like(w)

    zT = z_ref[...]
    nsq = jnp.sum(zT * zT, axis=0, keepdims=True)
    nzT = zT / jnp.maximum(jnp.sqrt(nsq), 1e-12)

    def chunk(c):
        return jax.lax.dot_general(
            ncb_ref[pl.ds(c * _KC, _KC), :], nzT,
            (((1,), (0,)), ((), ())),
            preferred_element_type=jnp.float32)

    acc_m = chunk(0)
    acc_c = jnp.zeros((_KC, TN), jnp.int32)
    for c in range(1, _NKC):
        dc = chunk(c)
        g = dc > acc_m
        acc_m = jnp.where(g, dc, acc_m)
        acc_c = jnp.where(g, c, acc_c)
    m = jnp.max(acc_m, axis=0, keepdims=True)
    srow = jax.lax.broadcasted_iota(jnp.int32, (_KC, TN), 0)
    cand = acc_c * _KC + srow
    idx = jnp.min(jnp.where(acc_m == m, cand, N_EMBED), axis=0,
                  keepdims=True)
    idx_ref[...] = idx.reshape(1, 1, TN)


def _tc_argmax(z2, weight):
    return pl.pallas_call(
        _argmax_body,
        grid=(GRID,),
        in_specs=[
            pl.BlockSpec((EMBED_DIM, TN), lambda i: (i, 0)),
            pl.BlockSpec((N_EMBED, EMBED_DIM), lambda i: (0, 0)),
        ],
        out_specs=[
            pl.BlockSpec((1, 1, TN), lambda i: (i, 0, 0)),
            pl.BlockSpec((N_EMBED, 2 * EMBED_DIM), lambda i: (0, 0)),
        ],
        out_shape=[
            jax.ShapeDtypeStruct((GRID, 1, TN), jnp.int32),
            jax.ShapeDtypeStruct((N_EMBED, 2 * EMBED_DIM), jnp.float32),
        ],
        scratch_shapes=[pltpu.VMEM((N_EMBED, EMBED_DIM), jnp.float32)],
    )(z2, weight)


_NC, _NS = 2, 16
_NW = _NC * _NS
_RPW = N_TOK // _NW
_CH = 128
_NCHUNK = _RPW // _CH


def _sc_gather_hist_body(w_hbm, idx_hbm, zq_hbm, cnt_hbm,
                         idx_v, rows_v, ones_v, zero_v, cnt_sh, sem):
    cid = lax.axis_index("c")
    sid = lax.axis_index("s")
    wid = cid * _NS + sid
    base = wid * _RPW

    for i in range(_CH // 16):
        ones_v[pl.ds(i * 16, 16)] = jnp.ones((16,), jnp.float32)
    for i in range((N_EMBED // _NS) // 16):
        zero_v[pl.ds(i * 16, 16)] = jnp.zeros((16,), jnp.float32)

    pltpu.sync_copy(zero_v, cnt_sh.at[pl.ds(sid * (N_EMBED // _NS),
                                            N_EMBED // _NS)])

    pltpu.sync_copy(idx_hbm.at[pl.ds(wid * _NCHUNK, _NCHUNK)], idx_v)

    plsc.subcore_barrier()

    for j in range(_NCHUNK):
        chunk_idx = idx_v.at[j]
        pltpu.async_copy(w_hbm.at[chunk_idx], rows_v, sem).wait()
        pltpu.sync_copy(rows_v, zq_hbm.at[pl.ds(base + j * _CH, _CH)])
        pltpu.sync_copy(ones_v, cnt_sh.at[chunk_idx], add=True)

    plsc.subcore_barrier()

    @pl.when(sid == 0)
    def _():
        pltpu.sync_copy(cnt_sh, cnt_hbm.at[pl.ds(cid * N_EMBED, N_EMBED)])


@functools.cache
def _sc_gather_hist():
    mesh = plsc.VectorSubcoreMesh(
        core_axis_name="c", subcore_axis_name="s",
        num_cores=_NC, num_subcores=_NS)
    return pl.kernel(
        _sc_gather_hist_body,
        out_type=(
            jax.ShapeDtypeStruct((N_TOK, 2 * EMBED_DIM), jnp.float32),
            jax.ShapeDtypeStruct((_NC * N_EMBED,), jnp.float32),
        ),
        mesh=mesh,
        scratch_types=[
            pltpu.VMEM((_NCHUNK, _CH), jnp.int32),
            pltpu.VMEM((_CH, 2 * EMBED_DIM), jnp.float32),
            pltpu.VMEM((_CH,), jnp.float32),
            pltpu.VMEM((N_EMBED // _NS,), jnp.float32),
            pltpu.VMEM_SHARED((N_EMBED,), jnp.float32),
            pltpu.SemaphoreType.DMA,
        ],
    )


_BB = N_TOK // 8


def _stats_body(zq_ref, z_ref, cnt_ref, zqt_ref, loss_ref, perp_ref,
                lacc_ref):
    step = pl.program_id(0)

    @pl.when(step == 0)
    def _():
        lacc_ref[0, 0] = 0.0

    zq = zq_ref[:, :EMBED_DIM]
    zqt = jnp.transpose(zq, (1, 0))
    zqt_ref[...] = zqt.reshape(1, EMBED_DIM, _BB)
    diff = zqt - z_ref[...]
    lacc_ref[0, 0] += jnp.sum(diff * diff)

    @pl.when(step == 7)
    def _():
        loss_ref[0, 0] = (1.0 + BETA) * lacc_ref[0, 0] / float(
            N_TOK * EMBED_DIM)
        probs = (cnt_ref[0:1, :] + cnt_ref[1:2, :]) / float(N_TOK)
        perp_ref[0, 0] = jnp.exp(-jnp.sum(probs * jnp.log(probs + 1e-10)))


def _tc_stats(zq_pad, z_flat, counts2):
    return pl.pallas_call(
        _stats_body,
        grid=(8,),
        in_specs=[
            pl.BlockSpec((_BB, 2 * EMBED_DIM), lambda i: (i, 0)),
            pl.BlockSpec((EMBED_DIM, _BB), lambda i: (i, 0)),
            pl.BlockSpec((2, N_EMBED), lambda i: (0, 0)),
        ],
        out_specs=[
            pl.BlockSpec((1, EMBED_DIM, _BB), lambda i: (i, 0, 0)),
            pl.BlockSpec(memory_space=pltpu.SMEM),
            pl.BlockSpec(memory_space=pltpu.SMEM),
        ],
        out_shape=[
            jax.ShapeDtypeStruct((8, EMBED_DIM, _BB), jnp.float32),
            jax.ShapeDtypeStruct((1, 1), jnp.float32),
            jax.ShapeDtypeStruct((1, 1), jnp.float32),
        ],
        scratch_shapes=[pltpu.SMEM((1, 1), jnp.float32)],
    )(zq_pad, z_flat, counts2)


def kernel(z, weight):
    z2 = z.reshape(8 * EMBED_DIM, _BB)
    idx3, wpad = _tc_argmax(z2, weight)
    idx_rows = idx3.reshape(N_TOK // _CH, _CH)
    zq_pad, cnt_flat = _sc_gather_hist()(wpad, idx_rows)
    counts2 = cnt_flat.reshape(_NC, N_EMBED)
    zqt, loss, perp = _tc_stats(zq_pad, z2, counts2)
    z_q_out = zqt.reshape(z.shape)
    return (z_q_out, idx3.reshape(N_TOK), loss[0, 0], perp[0, 0])


# speedup vs baseline: 1.0578x; 1.0578x over previous
import functools

import jax
import jax.numpy as jnp
from jax import lax
from jax.experimental import pallas as pl
from jax.experimental.pallas import tpu as pltpu
from jax.experimental.pallas import tpu_sc as plsc

N_TOK = 8192
N_EMBED = 8192
EMBED_DIM = 64
BETA = 0.25
TN = 1024
GRID = N_TOK // TN


_KC = 256
_NKC = N_EMBED // _KC


def _argmax_body(z_ref, w_ref, idx_ref, wpad_ref, ncb_ref):
    step = pl.program_id(0)

    @pl.when(step == 0)
    def _():
        w = w_ref[...]
        ncb_ref[...] = w / jnp.maximum(
            jnp.sqrt(jnp.sum(w * w, axis=1, keepdims=True)), 1e-12)
        wpad_ref[:, :EMBED_DIM] = w
        wpad_ref[:, EMBED_DIM:] = jnp.zeros_like(w)

    zt = z_ref[...]
    nz = zt / jnp.maximum(
        jnp.sqrt(jnp.sum(zt * zt, axis=1, keepdims=True)), 1e-12)

    def chunk(c):
        return jax.lax.dot_general(
            nz, ncb_ref[pl.ds(c * _KC, _KC), :],
            (((1,), (1,)), ((), ())),
            preferred_element_type=jnp.float32)

    acc_m = chunk(0)
    acc_c = jnp.zeros((TN, _KC), jnp.int32)
    for c in range(1, _NKC):
        dc = chunk(c)
        g = dc > acc_m
        acc_m = jnp.where(g, dc, acc_m)
        acc_c = jnp.where(g, c, acc_c)
    m = jnp.max(acc_m, axis=1, keepdims=True)
    lane = jax.lax.broadcasted_iota(jnp.int32, (TN, _KC), 1)
    cand = acc_c * _KC + lane
    idx_ref[...] = jnp.min(jnp.where(acc_m == m, cand, N_EMBED), axis=1,
                           keepdims=True)


def _tc_argmax(z_flat, weight):
    return pl.pallas_call(
        _argmax_body,
        grid=(GRID,),
        in_specs=[
            pl.BlockSpec((TN, EMBED_DIM), lambda i: (i, 0)),
            pl.BlockSpec((N_EMBED, EMBED_DIM), lambda i: (0, 0)),
        ],
        out_specs=[
            pl.BlockSpec((TN, 1), lambda i: (i, 0)),
            pl.BlockSpec((N_EMBED, 2 * EMBED_DIM), lambda i: (0, 0)),
        ],
        out_shape=[
            jax.ShapeDtypeStruct((N_TOK, 1), jnp.int32),
            jax.ShapeDtypeStruct((N_EMBED, 2 * EMBED_DIM), jnp.float32),
        ],
        scratch_shapes=[pltpu.VMEM((N_EMBED, EMBED_DIM), jnp.float32)],
    )(z_flat, weight)


_NC, _NS = 2, 16
_NW = _NC * _NS
_RPW = N_TOK // _NW
_CH = 128
_NCHUNK = _RPW // _CH


def _sc_gather_hist_body(w_hbm, idx_hbm, zq_hbm, cnt_hbm,
                         idx_v, rows_v, ones_v, zero_v, cnt_sh, sem):
    cid = lax.axis_index("c")
    sid = lax.axis_index("s")
    wid = cid * _NS + sid
    base = wid * _RPW

    for i in range(_CH // 16):
        ones_v[pl.ds(i * 16, 16)] = jnp.ones((16,), jnp.float32)
    for i in range((N_EMBED // _NS) // 16):
        zero_v[pl.ds(i * 16, 16)] = jnp.zeros((16,), jnp.float32)

    pltpu.sync_copy(zero_v, cnt_sh.at[pl.ds(sid * (N_EMBED // _NS),
                                            N_EMBED // _NS)])

    pltpu.sync_copy(idx_hbm.at[pl.ds(wid * _NCHUNK, _NCHUNK)], idx_v)

    plsc.subcore_barrier()

    for j in range(_NCHUNK):
        chunk_idx = idx_v.at[j]
        pltpu.async_copy(w_hbm.at[chunk_idx], rows_v, sem).wait()
        pltpu.sync_copy(rows_v, zq_hbm.at[pl.ds(base + j * _CH, _CH)])
        pltpu.sync_copy(ones_v, cnt_sh.at[chunk_idx], add=True)

    plsc.subcore_barrier()

    @pl.when(sid == 0)
    def _():
        pltpu.sync_copy(cnt_sh, cnt_hbm.at[pl.ds(cid * N_EMBED, N_EMBED)])


@functools.cache
def _sc_gather_hist():
    mesh = plsc.VectorSubcoreMesh(
        core_axis_name="c", subcore_axis_name="s",
        num_cores=_NC, num_subcores=_NS)
    return pl.kernel(
        _sc_gather_hist_body,
        out_type=(
            jax.ShapeDtypeStruct((N_TOK, 2 * EMBED_DIM), jnp.float32),
            jax.ShapeDtypeStruct((_NC * N_EMBED,), jnp.float32),
        ),
        mesh=mesh,
        scratch_types=[
            pltpu.VMEM((_NCHUNK, _CH), jnp.int32),
            pltpu.VMEM((_CH, 2 * EMBED_DIM), jnp.float32),
            pltpu.VMEM((_CH,), jnp.float32),
            pltpu.VMEM((N_EMBED // _NS,), jnp.float32),
            pltpu.VMEM_SHARED((N_EMBED,), jnp.float32),
            pltpu.SemaphoreType.DMA,
        ],
    )


_BB = N_TOK // 8


def _stats_body(zq_ref, z_ref, cnt_ref, zqt_ref, loss_ref, perp_ref,
                lacc_ref):
    step = pl.program_id(0)

    @pl.when(step == 0)
    def _():
        lacc_ref[0, 0] = 0.0

    zq = zq_ref[:, :EMBED_DIM]
    zqt = jnp.transpose(zq, (1, 0))
    zqt_ref[...] = zqt.reshape(1, EMBED_DIM, _BB)
    diff = zq - z_ref[...]
    lacc_ref[0, 0] += jnp.sum(diff * diff)

    @pl.when(step == 7)
    def _():
        loss_ref[0, 0] = (1.0 + BETA) * lacc_ref[0, 0] / float(
            N_TOK * EMBED_DIM)
        probs = (cnt_ref[0:1, :] + cnt_ref[1:2, :]) / float(N_TOK)
        perp_ref[0, 0] = jnp.exp(-jnp.sum(probs * jnp.log(probs + 1e-10)))


def _tc_stats(zq_pad, z_flat, counts2):
    return pl.pallas_call(
        _stats_body,
        grid=(8,),
        in_specs=[
            pl.BlockSpec((_BB, 2 * EMBED_DIM), lambda i: (i, 0)),
            pl.BlockSpec((_BB, EMBED_DIM), lambda i: (i, 0)),
            pl.BlockSpec((2, N_EMBED), lambda i: (0, 0)),
        ],
        out_specs=[
            pl.BlockSpec((1, EMBED_DIM, _BB), lambda i: (i, 0, 0)),
            pl.BlockSpec(memory_space=pltpu.SMEM),
            pl.BlockSpec(memory_space=pltpu.SMEM),
        ],
        out_shape=[
            jax.ShapeDtypeStruct((8, EMBED_DIM, _BB), jnp.float32),
            jax.ShapeDtypeStruct((1, 1), jnp.float32),
            jax.ShapeDtypeStruct((1, 1), jnp.float32),
        ],
        scratch_shapes=[pltpu.SMEM((1, 1), jnp.float32)],
    )(zq_pad, z_flat, counts2)


def kernel(z, weight):
    zt = jnp.transpose(z, (0, 2, 3, 1))
    z_flat = zt.reshape(-1, EMBED_DIM)
    idx2, wpad = _tc_argmax(z_flat, weight)
    idx_rows = idx2.reshape(N_TOK // _CH, _CH)
    zq_pad, cnt_flat = _sc_gather_hist()(wpad, idx_rows)
    counts2 = cnt_flat.reshape(_NC, N_EMBED)
    zqt, loss, perp = _tc_stats(zq_pad, z_flat, counts2)
    z_q_out = zqt.reshape(z.shape)
    return (z_q_out, idx2.reshape(N_TOK), loss[0, 0], perp[0, 0])


# speedup vs baseline: 1.0668x; 1.0085x over previous
import functools

import jax
import jax.numpy as jnp
from jax import lax
from jax.experimental import pallas as pl
from jax.experimental.pallas import tpu as pltpu
from jax.experimental.pallas import tpu_sc as plsc

N_TOK = 8192
N_EMBED = 8192
EMBED_DIM = 64
BETA = 0.25
TN = 2048
GRID = N_TOK // TN


_KC = 256
_NKC = N_EMBED // _KC


def _argmax_body(z_ref, w_ref, idx_ref, wpad_ref, ncb_ref):
    step = pl.program_id(0)

    @pl.when(step == 0)
    def _():
        w = w_ref[...]
        ncb_ref[...] = w / jnp.maximum(
            jnp.sqrt(jnp.sum(w * w, axis=1, keepdims=True)), 1e-12)
        wpad_ref[:, :EMBED_DIM] = w
        wpad_ref[:, EMBED_DIM:] = jnp.zeros_like(w)

    zt = z_ref[...]
    nz = zt / jnp.maximum(
        jnp.sqrt(jnp.sum(zt * zt, axis=1, keepdims=True)), 1e-12)

    def chunk(c):
        return jax.lax.dot_general(
            nz, ncb_ref[pl.ds(c * _KC, _KC), :],
            (((1,), (1,)), ((), ())),
            preferred_element_type=jnp.float32)

    acc_m = chunk(0)
    acc_c = jnp.zeros((TN, _KC), jnp.int32)
    for c in range(1, _NKC):
        dc = chunk(c)
        g = dc > acc_m
        acc_m = jnp.where(g, dc, acc_m)
        acc_c = jnp.where(g, c, acc_c)
    m = jnp.max(acc_m, axis=1, keepdims=True)
    lane = jax.lax.broadcasted_iota(jnp.int32, (TN, _KC), 1)
    cand = acc_c * _KC + lane
    idx_ref[...] = jnp.min(jnp.where(acc_m == m, cand, N_EMBED), axis=1,
                           keepdims=True)


def _tc_argmax(z_flat, weight):
    return pl.pallas_call(
        _argmax_body,
        grid=(GRID,),
        in_specs=[
            pl.BlockSpec((TN, EMBED_DIM), lambda i: (i, 0)),
            pl.BlockSpec((N_EMBED, EMBED_DIM), lambda i: (0, 0)),
        ],
        out_specs=[
            pl.BlockSpec((TN, 1), lambda i: (i, 0)),
            pl.BlockSpec((N_EMBED, 2 * EMBED_DIM), lambda i: (0, 0)),
        ],
        out_shape=[
            jax.ShapeDtypeStruct((N_TOK, 1), jnp.int32),
            jax.ShapeDtypeStruct((N_EMBED, 2 * EMBED_DIM), jnp.float32),
        ],
        scratch_shapes=[pltpu.VMEM((N_EMBED, EMBED_DIM), jnp.float32)],
    )(z_flat, weight)


_NC, _NS = 2, 16
_NW = _NC * _NS
_RPW = N_TOK // _NW
_CH = 128
_NCHUNK = _RPW // _CH


def _sc_gather_hist_body(w_hbm, idx_hbm, zq_hbm, cnt_hbm,
                         idx_v, rows_v, ones_v, zero_v, cnt_sh, sem):
    cid = lax.axis_index("c")
    sid = lax.axis_index("s")
    wid = cid * _NS + sid
    base = wid * _RPW

    for i in range(_CH // 16):
        ones_v[pl.ds(i * 16, 16)] = jnp.ones((16,), jnp.float32)
    for i in range((N_EMBED // _NS) // 16):
        zero_v[pl.ds(i * 16, 16)] = jnp.zeros((16,), jnp.float32)

    pltpu.sync_copy(zero_v, cnt_sh.at[pl.ds(sid * (N_EMBED // _NS),
                                            N_EMBED // _NS)])

    pltpu.sync_copy(idx_hbm.at[pl.ds(wid * _NCHUNK, _NCHUNK)], idx_v)

    plsc.subcore_barrier()

    for j in range(_NCHUNK):
        chunk_idx = idx_v.at[j]
        pltpu.async_copy(w_hbm.at[chunk_idx], rows_v, sem).wait()
        pltpu.sync_copy(rows_v, zq_hbm.at[pl.ds(base + j * _CH, _CH)])
        pltpu.sync_copy(ones_v, cnt_sh.at[chunk_idx], add=True)

    plsc.subcore_barrier()

    @pl.when(sid == 0)
    def _():
        pltpu.sync_copy(cnt_sh, cnt_hbm.at[pl.ds(cid * N_EMBED, N_EMBED)])


@functools.cache
def _sc_gather_hist():
    mesh = plsc.VectorSubcoreMesh(
        core_axis_name="c", subcore_axis_name="s",
        num_cores=_NC, num_subcores=_NS)
    return pl.kernel(
        _sc_gather_hist_body,
        out_type=(
            jax.ShapeDtypeStruct((N_TOK, 2 * EMBED_DIM), jnp.float32),
            jax.ShapeDtypeStruct((_NC * N_EMBED,), jnp.float32),
        ),
        mesh=mesh,
        scratch_types=[
            pltpu.VMEM((_NCHUNK, _CH), jnp.int32),
            pltpu.VMEM((_CH, 2 * EMBED_DIM), jnp.float32),
            pltpu.VMEM((_CH,), jnp.float32),
            pltpu.VMEM((N_EMBED // _NS,), jnp.float32),
            pltpu.VMEM_SHARED((N_EMBED,), jnp.float32),
            pltpu.SemaphoreType.DMA,
        ],
    )


_BB = N_TOK // 8


def _stats_body(zq_ref, z_ref, cnt_ref, zqt_ref, loss_ref, perp_ref,
                lacc_ref):
    step = pl.program_id(0)

    @pl.when(step == 0)
    def _():
        lacc_ref[0, 0] = 0.0

    zq = zq_ref[:, :EMBED_DIM]
    zqt = jnp.transpose(zq, (1, 0))
    zqt_ref[...] = zqt.reshape(1, EMBED_DIM, _BB)
    diff = zq - z_ref[...]
    lacc_ref[0, 0] += jnp.sum(diff * diff)

    @pl.when(step == 7)
    def _():
        loss_ref[0, 0] = (1.0 + BETA) * lacc_ref[0, 0] / float(
            N_TOK * EMBED_DIM)
        probs = (cnt_ref[0:1, :] + cnt_ref[1:2, :]) / float(N_TOK)
        perp_ref[0, 0] = jnp.exp(-jnp.sum(probs * jnp.log(probs + 1e-10)))


def _tc_stats(zq_pad, z_flat, counts2):
    return pl.pallas_call(
        _stats_body,
        grid=(8,),
        in_specs=[
            pl.BlockSpec((_BB, 2 * EMBED_DIM), lambda i: (i, 0)),
            pl.BlockSpec((_BB, EMBED_DIM), lambda i: (i, 0)),
            pl.BlockSpec((2, N_EMBED), lambda i: (0, 0)),
        ],
        out_specs=[
            pl.BlockSpec((1, EMBED_DIM, _BB), lambda i: (i, 0, 0)),
            pl.BlockSpec(memory_space=pltpu.SMEM),
            pl.BlockSpec(memory_space=pltpu.SMEM),
        ],
        out_shape=[
            jax.ShapeDtypeStruct((8, EMBED_DIM, _BB), jnp.float32),
            jax.ShapeDtypeStruct((1, 1), jnp.float32),
            jax.ShapeDtypeStruct((1, 1), jnp.float32),
        ],
        scratch_shapes=[pltpu.SMEM((1, 1), jnp.float32)],
    )(zq_pad, z_flat, counts2)


def kernel(z, weight):
    zt = jnp.transpose(z, (0, 2, 3, 1))
    z_flat = zt.reshape(-1, EMBED_DIM)
    idx2, wpad = _tc_argmax(z_flat, weight)
    idx_rows = idx2.reshape(N_TOK // _CH, _CH)
    zq_pad, cnt_flat = _sc_gather_hist()(wpad, idx_rows)
    counts2 = cnt_flat.reshape(_NC, N_EMBED)
    zqt, loss, perp = _tc_stats(zq_pad, z_flat, counts2)
    z_q_out = zqt.reshape(z.shape)
    return (z_q_out, idx2.reshape(N_TOK), loss[0, 0], perp[0, 0])


# speedup vs baseline: 1.0701x; 1.0031x over previous
import functools

import jax
import jax.numpy as jnp
from jax import lax
from jax.experimental import pallas as pl
from jax.experimental.pallas import tpu as pltpu
from jax.experimental.pallas import tpu_sc as plsc

N_TOK = 8192
N_EMBED = 8192
EMBED_DIM = 64
BETA = 0.25
TN = 4096
GRID = N_TOK // TN


_KC = 256
_NKC = N_EMBED // _KC


def _argmax_body(z_ref, w_ref, idx_ref, wpad_ref, ncb_ref):
    step = pl.program_id(0)

    @pl.when(step == 0)
    def _():
        w = w_ref[...]
        ncb_ref[...] = w / jnp.maximum(
            jnp.sqrt(jnp.sum(w * w, axis=1, keepdims=True)), 1e-12)
        wpad_ref[:, :EMBED_DIM] = w
        wpad_ref[:, EMBED_DIM:] = jnp.zeros_like(w)

    zt = z_ref[...]
    nz = zt / jnp.maximum(
        jnp.sqrt(jnp.sum(zt * zt, axis=1, keepdims=True)), 1e-12)

    def chunk(c):
        return jax.lax.dot_general(
            nz, ncb_ref[pl.ds(c * _KC, _KC), :],
            (((1,), (1,)), ((), ())),
            preferred_element_type=jnp.float32)

    acc_m = chunk(0)
    acc_c = jnp.zeros((TN, _KC), jnp.int32)
    for c in range(1, _NKC):
        dc = chunk(c)
        g = dc > acc_m
        acc_m = jnp.where(g, dc, acc_m)
        acc_c = jnp.where(g, c, acc_c)
    m = jnp.max(acc_m, axis=1, keepdims=True)
    lane = jax.lax.broadcasted_iota(jnp.int32, (TN, _KC), 1)
    cand = acc_c * _KC + lane
    idx_ref[...] = jnp.min(jnp.where(acc_m == m, cand, N_EMBED), axis=1,
                           keepdims=True)


def _tc_argmax(z_flat, weight):
    return pl.pallas_call(
        _argmax_body,
        grid=(GRID,),
        in_specs=[
            pl.BlockSpec((TN, EMBED_DIM), lambda i: (i, 0)),
            pl.BlockSpec((N_EMBED, EMBED_DIM), lambda i: (0, 0)),
        ],
        out_specs=[
            pl.BlockSpec((TN, 1), lambda i: (i, 0)),
            pl.BlockSpec((N_EMBED, 2 * EMBED_DIM), lambda i: (0, 0)),
        ],
        out_shape=[
            jax.ShapeDtypeStruct((N_TOK, 1), jnp.int32),
            jax.ShapeDtypeStruct((N_EMBED, 2 * EMBED_DIM), jnp.float32),
        ],
        scratch_shapes=[pltpu.VMEM((N_EMBED, EMBED_DIM), jnp.float32)],
    )(z_flat, weight)


_NC, _NS = 2, 16
_NW = _NC * _NS
_RPW = N_TOK // _NW
_CH = 128
_NCHUNK = _RPW // _CH


def _sc_gather_hist_body(w_hbm, idx_hbm, zq_hbm, cnt_hbm,
                         idx_v, rows_v, ones_v, zero_v, cnt_sh, sem):
    cid = lax.axis_index("c")
    sid = lax.axis_index("s")
    wid = cid * _NS + sid
    base = wid * _RPW

    for i in range(_CH // 16):
        ones_v[pl.ds(i * 16, 16)] = jnp.ones((16,), jnp.float32)
    for i in range((N_EMBED // _NS) // 16):
        zero_v[pl.ds(i * 16, 16)] = jnp.zeros((16,), jnp.float32)

    pltpu.sync_copy(zero_v, cnt_sh.at[pl.ds(sid * (N_EMBED // _NS),
                                            N_EMBED // _NS)])

    pltpu.sync_copy(idx_hbm.at[pl.ds(wid * _NCHUNK, _NCHUNK)], idx_v)

    plsc.subcore_barrier()

    for j in range(_NCHUNK):
        chunk_idx = idx_v.at[j]
        pltpu.async_copy(w_hbm.at[chunk_idx], rows_v, sem).wait()
        pltpu.sync_copy(rows_v, zq_hbm.at[pl.ds(base + j * _CH, _CH)])
        pltpu.sync_copy(ones_v, cnt_sh.at[chunk_idx], add=True)

    plsc.subcore_barrier()

    @pl.when(sid == 0)
    def _():
        pltpu.sync_copy(cnt_sh, cnt_hbm.at[pl.ds(cid * N_EMBED, N_EMBED)])


@functools.cache
def _sc_gather_hist():
    mesh = plsc.VectorSubcoreMesh(
        core_axis_name="c", subcore_axis_name="s",
        num_cores=_NC, num_subcores=_NS)
    return pl.kernel(
        _sc_gather_hist_body,
        out_type=(
            jax.ShapeDtypeStruct((N_TOK, 2 * EMBED_DIM), jnp.float32),
            jax.ShapeDtypeStruct((_NC * N_EMBED,), jnp.float32),
        ),
        mesh=mesh,
        scratch_types=[
            pltpu.VMEM((_NCHUNK, _CH), jnp.int32),
            pltpu.VMEM((_CH, 2 * EMBED_DIM), jnp.float32),
            pltpu.VMEM((_CH,), jnp.float32),
            pltpu.VMEM((N_EMBED // _NS,), jnp.float32),
            pltpu.VMEM_SHARED((N_EMBED,), jnp.float32),
            pltpu.SemaphoreType.DMA,
        ],
    )


_BB = N_TOK // 8


def _stats_body(zq_ref, z_ref, cnt_ref, zqt_ref, loss_ref, perp_ref,
                lacc_ref):
    step = pl.program_id(0)

    @pl.when(step == 0)
    def _():
        lacc_ref[0, 0] = 0.0

    zq = zq_ref[:, :EMBED_DIM]
    zqt = jnp.transpose(zq, (1, 0))
    zqt_ref[...] = zqt.reshape(1, EMBED_DIM, _BB)
    diff = zq - z_ref[...]
    lacc_ref[0, 0] += jnp.sum(diff * diff)

    @pl.when(step == 7)
    def _():
        loss_ref[0, 0] = (1.0 + BETA) * lacc_ref[0, 0] / float(
            N_TOK * EMBED_DIM)
        probs = (cnt_ref[0:1, :] + cnt_ref[1:2, :]) / float(N_TOK)
        perp_ref[0, 0] = jnp.exp(-jnp.sum(probs * jnp.log(probs + 1e-10)))


def _tc_stats(zq_pad, z_flat, counts2):
    return pl.pallas_call(
        _stats_body,
        grid=(8,),
        in_specs=[
            pl.BlockSpec((_BB, 2 * EMBED_DIM), lambda i: (i, 0)),
            pl.BlockSpec((_BB, EMBED_DIM), lambda i: (i, 0)),
            pl.BlockSpec((2, N_EMBED), lambda i: (0, 0)),
        ],
        out_specs=[
            pl.BlockSpec((1, EMBED_DIM, _BB), lambda i: (i, 0, 0)),
            pl.BlockSpec(memory_space=pltpu.SMEM),
            pl.BlockSpec(memory_space=pltpu.SMEM),
        ],
        out_shape=[
            jax.ShapeDtypeStruct((8, EMBED_DIM, _BB), jnp.float32),
            jax.ShapeDtypeStruct((1, 1), jnp.float32),
            jax.ShapeDtypeStruct((1, 1), jnp.float32),
        ],
        scratch_shapes=[pltpu.SMEM((1, 1), jnp.float32)],
    )(zq_pad, z_flat, counts2)


def kernel(z, weight):
    zt = jnp.transpose(z, (0, 2, 3, 1))
    z_flat = zt.reshape(-1, EMBED_DIM)
    idx2, wpad = _tc_argmax(z_flat, weight)
    idx_rows = idx2.reshape(N_TOK // _CH, _CH)
    zq_pad, cnt_flat = _sc_gather_hist()(wpad, idx_rows)
    counts2 = cnt_flat.reshape(_NC, N_EMBED)
    zqt, loss, perp = _tc_stats(zq_pad, z_flat, counts2)
    z_q_out = zqt.reshape(z.shape)
    return (z_q_out, idx2.reshape(N_TOK), loss[0, 0], perp[0, 0])
